# Initial kernel scaffold; baseline (speedup 1.0000x reference)
import jax
import jax.numpy as jnp
from jax.experimental import pallas as pl


def kernel(pos_seq, pe_k):
    raise NotImplementedError("write your pallas kernel here")


# speedup vs baseline: 5.7417x; 5.7417x over previous
import functools

import jax
import jax.numpy as jnp
from jax import lax
from jax.experimental import pallas as pl
from jax.experimental.pallas import tpu as pltpu
from jax.experimental.pallas import tpu_sc as plsc

D_MODEL = 1024
MAXLEN = 8192
SEQ = 16383

NW = 32
SEQ_PAD = 16384
BW = SEQ_PAD // NW
K = 32
NCH = BW // K
LANES = 16

_mesh = plsc.VectorSubcoreMesh(core_axis_name="c", subcore_axis_name="s")


@functools.partial(
    pl.kernel,
    mesh=_mesh,
    out_type=jax.ShapeDtypeStruct((SEQ, D_MODEL), jnp.float32),
    scratch_types=[
        pltpu.VMEM((BW,), jnp.int32),
        pltpu.VMEM((NCH, K), jnp.int32),
        pltpu.VMEM((NCH, K), jnp.int32),
        pltpu.VMEM((K, D_MODEL), jnp.float32),
        pltpu.VMEM((K, D_MODEL), jnp.float32),
        pltpu.SemaphoreType.DMA,
        pltpu.SemaphoreType.DMA,
    ],
)
def _gather_rows(table_hbm, idx_hbm, out_hbm, idx_stage, idx2d, oidx2d,
                 buf0, buf1, sem0, sem1):
    wid = lax.axis_index("s") * 2 + lax.axis_index("c")
    base = wid * BW

    pltpu.sync_copy(idx_hbm.at[pl.ds(base, BW)], idx_stage)
    per_row = K // LANES
    lane = lax.iota(jnp.int32, LANES)
    for j in range(BW // LANES):
        v = idx_stage[pl.ds(j * LANES, LANES)]
        c = jnp.minimum(jnp.maximum(v, -MAXLEN), MAXLEN - 1) + MAXLEN
        idx2d[j // per_row, pl.ds((j % per_row) * LANES, LANES)] = c
        o = jnp.minimum(base + j * LANES + lane, SEQ - 1)
        oidx2d[j // per_row, pl.ds((j % per_row) * LANES, LANES)] = o

    bufs = (buf0, buf1)
    sems = (sem0, sem1)
    copies = [None] * NCH

    def start(g):
        copies[g] = pltpu.async_copy(
            table_hbm.at[idx2d.at[g]], bufs[g % 2], sems[g % 2])

    start(0)
    for g in range(NCH):
        if g + 1 < NCH:
            start(g + 1)
        copies[g].wait()
        pltpu.sync_copy(bufs[g % 2], out_hbm.at[oidx2d.at[g]])


def kernel(pos_seq, pe_k):
    ps = jnp.concatenate([pos_seq, pos_seq[-1:]]).astype(jnp.int32)
    out_k = _gather_rows(pe_k, ps)
    return (out_k, None)


# speedup vs baseline: 5.7785x; 1.0064x over previous
import functools

import jax
import jax.numpy as jnp
from jax import lax
from jax.experimental import pallas as pl
from jax.experimental.pallas import tpu as pltpu
from jax.experimental.pallas import tpu_sc as plsc

D_MODEL = 1024
MAXLEN = 8192
SEQ = 16383

NW = 32
SEQ_PAD = 16384
BW = SEQ_PAD // NW
K = 32
NCH = BW // K
LANES = 16

_mesh = plsc.VectorSubcoreMesh(core_axis_name="c", subcore_axis_name="s")


@functools.partial(
    pl.kernel,
    mesh=_mesh,
    out_type=jax.ShapeDtypeStruct((SEQ, D_MODEL), jnp.float32),
    scratch_types=[
        pltpu.VMEM((BW,), jnp.int32),
        pltpu.VMEM((NCH, K), jnp.int32),
        pltpu.VMEM((NCH, K), jnp.int32),
        pltpu.VMEM((K, D_MODEL), jnp.float32),
        pltpu.VMEM((K, D_MODEL), jnp.float32),
        pltpu.VMEM((K, D_MODEL), jnp.float32),
        pltpu.SemaphoreType.DMA,
        pltpu.SemaphoreType.DMA,
        pltpu.SemaphoreType.DMA,
        pltpu.SemaphoreType.DMA,
        pltpu.SemaphoreType.DMA,
        pltpu.SemaphoreType.DMA,
    ],
)
def _gather_rows(table_hbm, idx_hbm, out_hbm, idx_stage, idx2d, oidx2d,
                 buf0, buf1, buf2, gsem0, gsem1, gsem2, ssem0, ssem1, ssem2):
    wid = lax.axis_index("s") * 2 + lax.axis_index("c")
    base = wid * BW

    pltpu.sync_copy(idx_hbm.at[pl.ds(base, BW)], idx_stage)
    per_row = K // LANES
    lane = lax.iota(jnp.int32, LANES)
    for j in range(BW // LANES):
        v = idx_stage[pl.ds(j * LANES, LANES)]
        c = jnp.minimum(jnp.maximum(v, -MAXLEN), MAXLEN - 1) + MAXLEN
        idx2d[j // per_row, pl.ds((j % per_row) * LANES, LANES)] = c
        o = jnp.minimum(base + j * LANES + lane, SEQ - 1)
        oidx2d[j // per_row, pl.ds((j % per_row) * LANES, LANES)] = o

    bufs = (buf0, buf1, buf2)
    gsems = (gsem0, gsem1, gsem2)
    ssems = (ssem0, ssem1, ssem2)
    gathers = [None] * NCH
    scatters = [None] * NCH

    def start_gather(g):
        gathers[g] = pltpu.async_copy(
            table_hbm.at[idx2d.at[g]], bufs[g % 3], gsems[g % 3])

    def start_scatter(g):
        scatters[g] = pltpu.async_copy(
            bufs[g % 3], out_hbm.at[oidx2d.at[g]], ssems[g % 3])

    start_gather(0)
    start_gather(1)
    for g in range(NCH):
        gathers[g].wait()
        start_scatter(g)
        nxt = g + 2
        if nxt < NCH:
            if nxt - 3 >= 0:
                scatters[nxt - 3].wait()
            start_gather(nxt)
    for g in range(max(0, NCH - 3), NCH):
        scatters[g].wait()


def kernel(pos_seq, pe_k):
    ps = jnp.concatenate([pos_seq, pos_seq[-1:]]).astype(jnp.int32)
    out_k = _gather_rows(pe_k, ps)
    return (out_k, None)


# speedup vs baseline: 5.8256x; 1.0082x over previous
import functools

import jax
import jax.numpy as jnp
from jax import lax
from jax.experimental import pallas as pl
from jax.experimental.pallas import tpu as pltpu
from jax.experimental.pallas import tpu_sc as plsc

D_MODEL = 1024
MAXLEN = 8192
SEQ = 16383

NW = 32
SEQ_PAD = 16384
BW = SEQ_PAD // NW
K = 32
NCH = BW // K
LANES = 16

_mesh = plsc.VectorSubcoreMesh(core_axis_name="c", subcore_axis_name="s")


@functools.partial(
    pl.kernel,
    mesh=_mesh,
    out_type=jax.ShapeDtypeStruct((SEQ, D_MODEL), jnp.float32),
    scratch_types=[
        pltpu.VMEM((BW,), jnp.int32),
        pltpu.VMEM((NCH, K), jnp.int32),
        pltpu.VMEM((NCH, K), jnp.int32),
        pltpu.VMEM((K, D_MODEL), jnp.float32),
        pltpu.VMEM((K, D_MODEL), jnp.float32),
        pltpu.VMEM((K, D_MODEL), jnp.float32),
        pltpu.SemaphoreType.DMA,
        pltpu.SemaphoreType.DMA,
        pltpu.SemaphoreType.DMA,
        pltpu.SemaphoreType.DMA,
        pltpu.SemaphoreType.DMA,
        pltpu.SemaphoreType.DMA,
    ],
)
def _gather_rows(table_hbm, idx_hbm, out_hbm, idx_stage, idx2d, oidx2d,
                 buf0, buf1, buf2, gsem0, gsem1, gsem2, ssem0, ssem1, ssem2):
    wid = lax.axis_index("s") * 2 + lax.axis_index("c")
    base = wid * BW

    pltpu.sync_copy(idx_hbm.at[pl.ds(base, BW)], idx_stage)
    per_row = K // LANES
    lane = lax.iota(jnp.int32, LANES)
    for j in range(BW // LANES):
        v = idx_stage[pl.ds(j * LANES, LANES)]
        c = jnp.minimum(jnp.maximum(v, -MAXLEN), MAXLEN - 1) + MAXLEN
        idx2d[j // per_row, pl.ds((j % per_row) * LANES, LANES)] = c
        o = jnp.minimum(base + j * LANES + lane, SEQ - 1)
        oidx2d[j // per_row, pl.ds((j % per_row) * LANES, LANES)] = o

    bufs = (buf0, buf1, buf2)
    gsems = (gsem0, gsem1, gsem2)
    ssems = (ssem0, ssem1, ssem2)
    gathers = [None] * NCH
    scatters = [None] * NCH

    def start_gather(g):
        gathers[g] = pltpu.async_copy(
            table_hbm.at[idx2d.at[g]], bufs[g % 3], gsems[g % 3])

    def start_scatter(g):
        if g < NCH - 1:
            scatters[g] = pltpu.async_copy(
                bufs[g % 3], out_hbm.at[pl.ds(base + g * K, K)], ssems[g % 3])
        else:
            scatters[g] = pltpu.async_copy(
                bufs[g % 3], out_hbm.at[oidx2d.at[g]], ssems[g % 3])

    start_gather(0)
    start_gather(1)
    for g in range(NCH):
        gathers[g].wait()
        start_scatter(g)
        nxt = g + 2
        if nxt < NCH:
            if nxt - 3 >= 0:
                scatters[nxt - 3].wait()
            start_gather(nxt)
    for g in range(max(0, NCH - 3), NCH):
        scatters[g].wait()


def kernel(pos_seq, pe_k):
    ps = jnp.concatenate([pos_seq, pos_seq[-1:]]).astype(jnp.int32)
    out_k = _gather_rows(pe_k, ps)
    return (out_k, None)


# speedup vs baseline: 5.8767x; 1.0088x over previous
import functools

import jax
import jax.numpy as jnp
from jax import lax
from jax.experimental import pallas as pl
from jax.experimental.pallas import tpu as pltpu
from jax.experimental.pallas import tpu_sc as plsc

D_MODEL = 1024
MAXLEN = 8192
SEQ = 16383

NW = 32
SEQ_PAD = 16384
BW = SEQ_PAD // NW
K = 32
NCH = BW // K
LANES = 16

_mesh = plsc.VectorSubcoreMesh(core_axis_name="c", subcore_axis_name="s")


@functools.partial(
    pl.kernel,
    mesh=_mesh,
    out_type=jax.ShapeDtypeStruct((SEQ, D_MODEL), jnp.float32),
    scratch_types=[
        pltpu.VMEM((BW,), jnp.int32),
        pltpu.VMEM((NCH, K), jnp.int32),
        pltpu.VMEM((K, D_MODEL), jnp.float32),
        pltpu.VMEM((K, D_MODEL), jnp.float32),
        pltpu.VMEM((K, D_MODEL), jnp.float32),
        pltpu.SemaphoreType.DMA,
        pltpu.SemaphoreType.DMA,
        pltpu.SemaphoreType.DMA,
        pltpu.SemaphoreType.DMA,
        pltpu.SemaphoreType.DMA,
        pltpu.SemaphoreType.DMA,
    ],
)
def _gather_rows(table_hbm, idx_hbm, out_hbm, idx_stage, idx2d,
                 buf0, buf1, buf2, gsem0, gsem1, gsem2, ssem0, ssem1, ssem2):
    wid = lax.axis_index("s") * 2 + lax.axis_index("c")
    base = wid * BW
    is_last = wid == NW - 1

    @pl.when(is_last)
    def _():
        pltpu.sync_copy(idx_hbm.at[pl.ds(base, BW - 1)],
                        idx_stage.at[pl.ds(0, BW - 1)])

    @pl.when(jnp.logical_not(is_last))
    def _():
        pltpu.sync_copy(idx_hbm.at[pl.ds(base, BW)], idx_stage)

    per_row = K // LANES
    lane = lax.iota(jnp.int32, LANES)
    for j in range(BW // LANES):
        v = idx_stage[pl.ds(j * LANES, LANES)]
        c = jnp.minimum(jnp.maximum(v, -MAXLEN), MAXLEN - 1) + MAXLEN
        idx2d[j // per_row, pl.ds((j % per_row) * LANES, LANES)] = c

    bufs = (buf0, buf1, buf2)
    gsems = (gsem0, gsem1, gsem2)
    ssems = (ssem0, ssem1, ssem2)
    gathers = [None] * NCH
    scatters = [None] * NCH

    def start_gather(g):
        gathers[g] = pltpu.async_copy(
            table_hbm.at[idx2d.at[g]], bufs[g % 3], gsems[g % 3])

    def start_scatter(g):
        scatters[g] = pltpu.async_copy(
            bufs[g % 3], out_hbm.at[pl.ds(base + g * K, K)], ssems[g % 3])

    start_gather(0)
    start_gather(1)
    for g in range(NCH):
        gathers[g].wait()
        if g == NCH - 1:
            @pl.when(is_last)
            def _():
                b = bufs[g % 3]
                for l in range(D_MODEL // LANES):
                    b[K - 1, pl.ds(l * LANES, LANES)] = (
                        b[K - 2, pl.ds(l * LANES, LANES)])
                pltpu.sync_copy(b.at[pl.ds(0, 24)],
                                out_hbm.at[pl.ds(base + g * K, 24)])
                tail_ids = jnp.minimum(SEQ - (LANES - 1) + lane, SEQ - 1)
                pltpu.sync_copy(b.at[pl.ds(LANES, LANES)],
                                out_hbm.at[tail_ids])

            @pl.when(jnp.logical_not(is_last))
            def _():
                pltpu.sync_copy(bufs[g % 3], out_hbm.at[pl.ds(base + g * K, K)])
        else:
            start_scatter(g)
        nxt = g + 2
        if nxt < NCH:
            if nxt - 3 >= 0:
                scatters[nxt - 3].wait()
            start_gather(nxt)
    for g in range(NCH - 3, NCH - 1):
        scatters[g].wait()


def kernel(pos_seq, pe_k):
    out_k = _gather_rows(pe_k, pos_seq.astype(jnp.int32))
    return (out_k, None)
